# baseline (speedup 1.0000x reference)
import functools
import numpy as np

import jax
import jax.numpy as jnp
from jax import lax
from jax.experimental import pallas as pl
from jax.experimental.pallas import tpu as pltpu

_ROT1 = (13, 15, 26, 6)
_ROT2 = (17, 29, 16, 24)
_KS0 = np.int32(0)
_KS1 = np.int32(42)
_KS2 = np.int32(np.uint32(0) ^ np.uint32(42) ^ np.uint32(0x1BD11BDA))
_TINY = np.float32(np.finfo(np.float32).tiny)
_ONE_BITS = np.int32(0x3F800000)
_NEG_INF = np.float32(-np.inf)
_BIG_IDX = np.int32(2**31 - 1)


def _rotl(x, r):
    return lax.shift_left(x, np.int32(r)) | lax.shift_right_logical(
        x, np.int32(32 - r))


def _rounds(x0, x1, rots):
    for r in rots:
        x0 = x0 + x1
        x1 = _rotl(x1, r) ^ x0
    return x0, x1


def _threefry_bits(n):
    x0 = jnp.full_like(n, _KS0)
    x1 = n + _KS1
    x0, x1 = _rounds(x0, x1, _ROT1)
    x0 = x0 + _KS1
    x1 = x1 + np.int32(_KS2 + np.int32(1))
    x0, x1 = _rounds(x0, x1, _ROT2)
    x0 = x0 + _KS2
    x1 = x1 + np.int32(_KS0 + np.int32(2))
    x0, x1 = _rounds(x0, x1, _ROT1)
    x0 = x0 + _KS0
    x1 = x1 + np.int32(_KS1 + np.int32(3))
    x0, x1 = _rounds(x0, x1, _ROT2)
    x0 = x0 + _KS1
    x1 = x1 + np.int32(_KS2 + np.int32(4))
    x0, x1 = _rounds(x0, x1, _ROT1)
    x0 = x0 + _KS2
    x1 = x1 + np.int32(_KS0 + np.int32(5))
    return x0 ^ x1


def _sample_kernel(logits_ref, out_ref, best_val, best_idx, *, blk, cols,
                   nblk):
    i = pl.program_id(0)

    @pl.when(i == 0)
    def _init():
        best_val[...] = jnp.full_like(best_val, _NEG_INF)
        best_idx[...] = jnp.zeros_like(best_idx)

    rows = logits_ref.shape[0]
    base = i * blk
    col = base + lax.broadcasted_iota(jnp.int32, (rows, blk), 1)
    row = lax.broadcasted_iota(jnp.int32, (rows, blk), 0)
    n = row * np.int32(cols) + col

    bits = _threefry_bits(n)
    fbits = lax.shift_right_logical(bits, np.int32(9)) | _ONE_BITS
    u = lax.bitcast_convert_type(fbits, jnp.float32) - np.float32(1.0) + _TINY
    g = -jnp.log(-jnp.log(u))
    val = g + logits_ref[...]
    val = jnp.where(col < np.int32(cols), val, _NEG_INF)

    bmax = jnp.max(val, axis=1, keepdims=True)
    bidx = jnp.min(jnp.where(val == bmax, col, _BIG_IDX), axis=1,
                   keepdims=True)

    upd = bmax > best_val[...]
    best_val[...] = jnp.where(upd, bmax, best_val[...])
    best_idx[...] = jnp.where(upd, bidx, best_idx[...])

    @pl.when(i == nblk - 1)
    def _emit():
        out_ref[...] = best_idx[...]


def kernel(logits):
    rows, cols = logits.shape
    blk = 4096
    nblk = (cols + blk - 1) // blk
    out = pl.pallas_call(
        functools.partial(_sample_kernel, blk=blk, cols=cols, nblk=nblk),
        grid=(nblk,),
        in_specs=[pl.BlockSpec((rows, blk), lambda i: (0, i))],
        out_specs=pl.BlockSpec((rows, 1), lambda i: (0, 0)),
        out_shape=jax.ShapeDtypeStruct((rows, 1), jnp.int32),
        scratch_shapes=[
            pltpu.VMEM((rows, 1), jnp.float32),
            pltpu.VMEM((rows, 1), jnp.int32),
        ],
    )(logits)
    return out.reshape(-1)
